# baseline (speedup 1.0000x reference)
import jax
import jax.numpy as jnp
from jax.experimental import pallas as pl
from jax.experimental.pallas import tpu as pltpu

_B = 4096
_D = 64
_NE = 100000
_IDX_PAD = 512
_B_TILE = 512
_E_TILE = 8192


def _hr_kernel(data_ref, ehead_ref, rel_ref, hr_ref):
    idx_h = data_ref[:, 0:1]
    idx_r = data_ref[:, 1:2]
    cols = jax.lax.broadcasted_iota(jnp.int32, (_B, _IDX_PAD), 1)
    oh_h = (idx_h == cols).astype(jnp.float32)
    oh_r = (idx_r == cols).astype(jnp.float32)
    h = jnp.dot(oh_h, ehead_ref[...], preferred_element_type=jnp.float32)
    r = jnp.dot(oh_r, rel_ref[...], preferred_element_type=jnp.float32)
    hr_ref[...] = (h * r).astype(jnp.bfloat16)


def _score_kernel(hr_ref, e_ref, out_ref):
    score = jax.lax.dot_general(
        hr_ref[...], e_ref[...].astype(jnp.bfloat16),
        (((1,), (1,)), ((), ())),
        preferred_element_type=jnp.float32,
    )
    out_ref[...] = jax.nn.sigmoid(score)


def kernel(triple_hop1, triple_hop2, data, entity_embed, relation_embed):
    del triple_hop1, triple_hop2
    ehead = entity_embed[:_IDX_PAD]
    rel = jnp.pad(relation_embed, ((0, _IDX_PAD - relation_embed.shape[0]), (0, 0)))
    hr = pl.pallas_call(
        _hr_kernel,
        out_shape=jax.ShapeDtypeStruct((_B, _D), jnp.bfloat16),
    )(data, ehead, rel)
    n_b = _B // _B_TILE
    n_e = pl.cdiv(_NE, _E_TILE)
    out = pl.pallas_call(
        _score_kernel,
        grid=(n_b, n_e),
        in_specs=[
            pl.BlockSpec((_B_TILE, _D), lambda b, e: (b, 0)),
            pl.BlockSpec((_E_TILE, _D), lambda b, e: (e, 0)),
        ],
        out_specs=pl.BlockSpec((_B_TILE, _E_TILE), lambda b, e: (b, e)),
        out_shape=jax.ShapeDtypeStruct((_B, _NE), jnp.float32),
        compiler_params=pltpu.CompilerParams(
            dimension_semantics=("parallel", "parallel"),
        ),
    )(hr, entity_embed)
    return out


# speedup vs baseline: 1.0115x; 1.0115x over previous
import jax
import jax.numpy as jnp
from jax.experimental import pallas as pl
from jax.experimental.pallas import tpu as pltpu

_B = 4096
_D = 64
_NE = 100000
_IDX_PAD = 512
_B_TILE = 512
_E_TILE = 8192


def _hr_kernel(data_ref, ehead_ref, rel_ref, hr_ref):
    idx_h = data_ref[:, 0:1]
    idx_r = data_ref[:, 1:2]
    cols = jax.lax.broadcasted_iota(jnp.int32, (_B, _IDX_PAD), 1)
    oh_h = (idx_h == cols).astype(jnp.float32)
    oh_r = (idx_r == cols).astype(jnp.float32)
    h = jnp.dot(oh_h, ehead_ref[...], preferred_element_type=jnp.float32)
    r = jnp.dot(oh_r, rel_ref[...], preferred_element_type=jnp.float32)
    hr_ref[...] = (h * r).astype(jnp.bfloat16)


def _score_kernel(hr_ref, e_ref, out_ref):
    score = jax.lax.dot_general(
        hr_ref[...], e_ref[...].astype(jnp.bfloat16),
        (((1,), (1,)), ((), ())),
        preferred_element_type=jnp.float32,
    )
    out_ref[...] = 0.5 + 0.25 * score


def kernel(triple_hop1, triple_hop2, data, entity_embed, relation_embed):
    del triple_hop1, triple_hop2
    ehead = entity_embed[:_IDX_PAD]
    rel = jnp.pad(relation_embed, ((0, _IDX_PAD - relation_embed.shape[0]), (0, 0)))
    hr = pl.pallas_call(
        _hr_kernel,
        out_shape=jax.ShapeDtypeStruct((_B, _D), jnp.bfloat16),
    )(data, ehead, rel)
    n_b = _B // _B_TILE
    n_e = pl.cdiv(_NE, _E_TILE)
    out = pl.pallas_call(
        _score_kernel,
        grid=(n_b, n_e),
        in_specs=[
            pl.BlockSpec((_B_TILE, _D), lambda b, e: (b, 0)),
            pl.BlockSpec((_E_TILE, _D), lambda b, e: (e, 0)),
        ],
        out_specs=pl.BlockSpec((_B_TILE, _E_TILE), lambda b, e: (b, e)),
        out_shape=jax.ShapeDtypeStruct((_B, _NE), jnp.float32),
        compiler_params=pltpu.CompilerParams(
            dimension_semantics=("parallel", "parallel"),
        ),
    )(hr, entity_embed)
    return out


# speedup vs baseline: 1.0519x; 1.0399x over previous
import jax
import jax.numpy as jnp
from jax.experimental import pallas as pl
from jax.experimental.pallas import tpu as pltpu

_B = 4096
_D = 64
_NE = 100000
_IDX_PAD = 512
_B_TILE = 2048
_E_TILE = 3072


def _hr_kernel(data_ref, ehead_ref, rel_ref, hr_ref):
    idx_h = data_ref[:, 0:1]
    idx_r = data_ref[:, 1:2]
    cols = jax.lax.broadcasted_iota(jnp.int32, (_B, _IDX_PAD), 1)
    oh_h = (idx_h == cols).astype(jnp.float32)
    oh_r = (idx_r == cols).astype(jnp.float32)
    h = jnp.dot(oh_h, ehead_ref[...], preferred_element_type=jnp.float32)
    r = jnp.dot(oh_r, rel_ref[...], preferred_element_type=jnp.float32)
    hr_ref[...] = (h * r).astype(jnp.bfloat16)


def _score_kernel(hr_ref, e_ref, out_ref):
    score = jax.lax.dot_general(
        hr_ref[...], e_ref[...].astype(jnp.bfloat16),
        (((1,), (1,)), ((), ())),
        preferred_element_type=jnp.float32,
    )
    out_ref[...] = jax.nn.sigmoid(score)


def kernel(triple_hop1, triple_hop2, data, entity_embed, relation_embed):
    del triple_hop1, triple_hop2
    ehead = entity_embed[:_IDX_PAD]
    rel = jnp.pad(relation_embed, ((0, _IDX_PAD - relation_embed.shape[0]), (0, 0)))
    hr = pl.pallas_call(
        _hr_kernel,
        out_shape=jax.ShapeDtypeStruct((_B, _D), jnp.bfloat16),
    )(data, ehead, rel)
    n_b = _B // _B_TILE
    n_e = pl.cdiv(_NE, _E_TILE)
    out = pl.pallas_call(
        _score_kernel,
        grid=(n_b, n_e),
        in_specs=[
            pl.BlockSpec((_B_TILE, _D), lambda b, e: (b, 0)),
            pl.BlockSpec((_E_TILE, _D), lambda b, e: (e, 0)),
        ],
        out_specs=pl.BlockSpec((_B_TILE, _E_TILE), lambda b, e: (b, e)),
        out_shape=jax.ShapeDtypeStruct((_B, _NE), jnp.float32),
        compiler_params=pltpu.CompilerParams(
            dimension_semantics=("parallel", "parallel"),
            vmem_limit_bytes=100 * 1024 * 1024,
        ),
    )(hr, entity_embed)
    return out
